# baseline (speedup 1.0000x reference)
import functools

import jax
import jax.numpy as jnp
from jax import lax
from jax.experimental import pallas as pl
from jax.experimental.pallas import tpu as pltpu
from jax.experimental.pallas import tpu_sc as plsc

N_NODES = 10000
N_EDGES = 320000
D = 128
NP = 10240
NW = 32
GP = 128
NG = 2560
E_PAD = NG * GP
T = NG // NW
PAIRS = T // 2
RPT = NP // 16

_MESH = plsc.VectorSubcoreMesh(core_axis_name="c", subcore_axis_name="s")


def _sc_aggregate(xe, src, dst, z2d):
    @functools.partial(
        pl.kernel,
        mesh=_MESH,
        out_type=jax.ShapeDtypeStruct((2, NP, D), jnp.float32),
        scratch_types=[
            pltpu.VMEM((GP,), jnp.int32),
            pltpu.VMEM((GP,), jnp.int32),
            pltpu.VMEM((GP,), jnp.int32),
            pltpu.VMEM((GP,), jnp.int32),
            pltpu.VMEM((GP, D), jnp.float32),
            pltpu.VMEM((GP, D), jnp.float32),
            pltpu.VMEM_SHARED((NP, D), jnp.float32),
            pltpu.SemaphoreType.DMA,
            pltpu.SemaphoreType.DMA,
            pltpu.SemaphoreType.DMA,
            pltpu.SemaphoreType.DMA,
        ],
    )
    def agg(xe_hbm, src_hbm, dst_hbm, z2d_hbm, out_hbm,
            sidxA, didxA, sidxB, didxB, rowsA, rowsB, acc,
            isemA, isemB, gsemA, gsemB):
        core = lax.axis_index("c")
        tid = lax.axis_index("s")
        w = core * 16 + tid

        pltpu.sync_copy(z2d_hbm, rowsA)
        rbase = tid * RPT
        for i in range(RPT // GP):
            pltpu.sync_copy(rowsA, acc.at[pl.ds(rbase + i * GP, GP)])
        plsc.subcore_barrier()

        def base_of(k):
            return jnp.minimum(k * NW + w, NG - 1) * GP

        def start_idx(k, sidx, didx, isem):
            base = base_of(k)
            pltpu.async_copy(src_hbm.at[pl.ds(base, GP)], sidx, isem)
            pltpu.async_copy(dst_hbm.at[pl.ds(base, GP)], didx, isem)

        def wait_idx(sidx, didx, isem):
            pltpu.make_async_copy(src_hbm.at[pl.ds(0, GP)], sidx, isem).wait()
            pltpu.make_async_copy(dst_hbm.at[pl.ds(0, GP)], didx, isem).wait()

        def start_gather(sidx, rows, gsem):
            pltpu.async_copy(xe_hbm.at[sidx], rows, gsem)

        def wait_gather(sidx, rows, gsem):
            pltpu.make_async_copy(xe_hbm.at[sidx], rows, gsem).wait()

        start_idx(0, sidxA, didxA, isemA)
        wait_idx(sidxA, didxA, isemA)
        start_gather(sidxA, rowsA, gsemA)
        start_idx(1, sidxB, didxB, isemB)

        def body(p, carry):
            k = 2 * p
            wait_idx(sidxB, didxB, isemB)
            wait_gather(sidxA, rowsA, gsemA)
            start_gather(sidxB, rowsB, gsemB)
            pltpu.sync_copy(rowsA, acc.at[didxA], add=True)
            start_idx(k + 2, sidxA, didxA, isemA)
            wait_idx(sidxA, didxA, isemA)
            start_gather(sidxA, rowsA, gsemA)
            wait_gather(sidxB, rowsB, gsemB)
            pltpu.sync_copy(rowsB, acc.at[didxB], add=True)
            start_idx(k + 3, sidxB, didxB, isemB)
            return carry

        lax.fori_loop(0, PAIRS, body, 0)

        wait_gather(sidxA, rowsA, gsemA)
        wait_idx(sidxB, didxB, isemB)

        plsc.subcore_barrier()

        for i in range(RPT // GP):
            pltpu.sync_copy(acc.at[pl.ds(rbase + i * GP, GP)], rowsA)
            pltpu.sync_copy(rowsA, out_hbm.at[core, pl.ds(rbase + i * GP, GP)])

    return agg(xe, src, dst, z2d)


def _sc_count(dst, z2d, o2d):
    @functools.partial(
        pl.kernel,
        mesh=_MESH,
        out_type=jax.ShapeDtypeStruct((2, NP, D), jnp.float32),
        scratch_types=[
            pltpu.VMEM((GP,), jnp.int32),
            pltpu.VMEM((GP,), jnp.int32),
            pltpu.VMEM((GP, D), jnp.float32),
            pltpu.VMEM_SHARED((NP, D), jnp.float32),
            pltpu.SemaphoreType.DMA,
            pltpu.SemaphoreType.DMA,
        ],
    )
    def cnt_k(dst_hbm, z2d_hbm, o2d_hbm, out_hbm, didxA, didxB, rows, acc,
              isemA, isemB):
        core = lax.axis_index("c")
        tid = lax.axis_index("s")
        w = core * 16 + tid

        pltpu.sync_copy(z2d_hbm, rows)
        rbase = tid * RPT
        for i in range(RPT // GP):
            pltpu.sync_copy(rows, acc.at[pl.ds(rbase + i * GP, GP)])
        plsc.subcore_barrier()

        pltpu.sync_copy(o2d_hbm, rows)

        def base_of(k):
            return jnp.minimum(k * NW + w, NG - 1) * GP

        def start_idx(k, didx, isem):
            pltpu.async_copy(dst_hbm.at[pl.ds(base_of(k), GP)], didx, isem)

        def wait_idx(didx, isem):
            pltpu.make_async_copy(dst_hbm.at[pl.ds(0, GP)], didx, isem).wait()

        start_idx(0, didxA, isemA)

        def body(p, carry):
            k = 2 * p
            wait_idx(didxA, isemA)
            start_idx(k + 1, didxB, isemB)
            pltpu.sync_copy(rows, acc.at[didxA], add=True)
            wait_idx(didxB, isemB)
            start_idx(k + 2, didxA, isemA)
            pltpu.sync_copy(rows, acc.at[didxB], add=True)
            return carry

        lax.fori_loop(0, PAIRS, body, 0)
        wait_idx(didxA, isemA)

        plsc.subcore_barrier()

        for i in range(RPT // GP):
            pltpu.sync_copy(acc.at[pl.ds(rbase + i * GP, GP)], rows)
            pltpu.sync_copy(rows, out_hbm.at[core, pl.ds(rbase + i * GP, GP)])

    return cnt_k(dst, z2d, o2d)


def _tc_dense(xe, agg_part, cnt_part, W_l, W_r, b, relu):
    B = 1280

    def body(x_ref, a_ref, c_ref, wl_ref, wr_ref, b_ref, o_ref):
        a = a_ref[0] + a_ref[1]
        c = c_ref[0] + c_ref[1]
        mean = a / jnp.maximum(c, 1.0)
        dn = (((1,), (1,)), ((), ()))
        out = (lax.dot_general(mean, wl_ref[...], dn,
                               preferred_element_type=jnp.float32)
               + lax.dot_general(x_ref[...], wr_ref[...], dn,
                                 preferred_element_type=jnp.float32)
               + b_ref[...][None, :])
        if relu:
            out = jnp.maximum(out, 0.0)
        o_ref[...] = out

    return pl.pallas_call(
        body,
        grid=(NP // B,),
        in_specs=[
            pl.BlockSpec((B, D), lambda i: (i, 0)),
            pl.BlockSpec((2, B, D), lambda i: (0, i, 0)),
            pl.BlockSpec((2, B, D), lambda i: (0, i, 0)),
            pl.BlockSpec((D, D), lambda i: (0, 0)),
            pl.BlockSpec((D, D), lambda i: (0, 0)),
            pl.BlockSpec((D,), lambda i: (0,)),
        ],
        out_specs=pl.BlockSpec((B, D), lambda i: (i, 0)),
        out_shape=jax.ShapeDtypeStruct((NP, D), jnp.float32),
    )(xe, agg_part, cnt_part, W_l, W_r, b)


def kernel(x, edge_index, W1_l, W1_r, b1, W2_l, W2_r, b2):
    src = edge_index[0]
    dst = edge_index[1]

    npad = E_PAD - N_EDGES
    srcp = jnp.concatenate([src, jnp.zeros((npad,), jnp.int32)])
    dstp = jnp.concatenate([dst, jnp.full((npad,), N_NODES, jnp.int32)])

    xe = jnp.pad(x, ((0, NP - N_NODES), (0, 0)))
    z2d = jnp.zeros((GP, D), jnp.float32)
    o2d = jnp.ones((GP, D), jnp.float32)

    cnt = _sc_count(dstp, z2d, o2d)
    agg1 = _sc_aggregate(xe, srcp, dstp, z2d)
    h = _tc_dense(xe, agg1, cnt, W1_l, W1_r, b1, relu=True)
    agg2 = _sc_aggregate(h, srcp, dstp, z2d)
    out = _tc_dense(h, agg2, cnt, W2_l, W2_r, b2, relu=False)
    return out[:N_NODES]
